# Initial kernel scaffold; baseline (speedup 1.0000x reference)
import jax
import jax.numpy as jnp
from jax.experimental import pallas as pl


def kernel(x, table):
    raise NotImplementedError("write your pallas kernel here")


# speedup vs baseline: 1.3733x; 1.3733x over previous
import functools

import jax
import jax.numpy as jnp
from jax import lax
from jax.experimental import pallas as pl
from jax.experimental.pallas import tpu as pltpu
from jax.experimental.pallas import tpu_sc as plsc

_HASH_MOD = 1000001
_DIM = 32
_LANES = 16
_ROW = 128


@functools.cache
def _build(n_rows_total: int, dim: int):
    info = plsc.get_sparse_core_info()
    nc, ns = info.num_cores, info.num_subcores
    nw = nc * ns
    assert n_rows_total % nw == 0
    rows_per_w = n_rows_total // nw
    mesh = plsc.VectorSubcoreMesh(core_axis_name="c", subcore_axis_name="s")

    @functools.partial(
        pl.kernel,
        out_type=jax.ShapeDtypeStruct((n_rows_total * _ROW, dim), jnp.float32),
        mesh=mesh,
        compiler_params=pltpu.CompilerParams(use_tc_tiling_on_sc=False),
        scratch_types=[
            pltpu.VMEM((rows_per_w, _ROW), jnp.int32),
            pltpu.VMEM((_ROW, dim), jnp.float32),
            pltpu.SemaphoreType.DMA,
        ],
    )
    def k(x_hbm, table_hbm, out_hbm, idx_v, rows_v, gsem):
        wid = lax.axis_index("s") * nc + lax.axis_index("c")
        row0 = wid * rows_per_w
        pltpu.sync_copy(x_hbm.at[pl.ds(row0, rows_per_w)], idx_v)

        def mod_body(r, carry):
            for i in range(_ROW // _LANES):
                sl = pl.ds(i * _LANES, _LANES)
                v = idx_v[r, sl]
                idx_v[r, sl] = lax.rem(v, lax.full_like(v, _HASH_MOD))
            return carry

        lax.fori_loop(0, rows_per_w, mod_body, 0)

        def gather_body(r, carry):
            pltpu.async_copy(table_hbm.at[idx_v.at[r]], rows_v, gsem).wait()
            pltpu.sync_copy(rows_v, out_hbm.at[pl.ds((row0 + r) * _ROW, _ROW)])
            return carry

        lax.fori_loop(0, rows_per_w, gather_body, 0)

    return k


def kernel(x, table):
    n_total = x.size
    x2 = x.reshape(n_total // _ROW, _ROW)
    out = _build(n_total // _ROW, table.shape[1])(x2, table)
    return out.reshape(*x.shape, table.shape[1])


# speedup vs baseline: 1.5563x; 1.1332x over previous
import functools

import jax
import jax.numpy as jnp
from jax import lax
from jax.experimental import pallas as pl
from jax.experimental.pallas import tpu as pltpu
from jax.experimental.pallas import tpu_sc as plsc

_HASH_MOD = 1000001
_DIM = 32
_LANES = 16
_CHUNK = 1664
_PHASES = 8


@functools.cache
def _build(n_total: int, dim: int):
    info = plsc.get_sparse_core_info()
    nc, ns = info.num_cores, info.num_subcores
    nw = nc * ns
    assert n_total % nw == 0
    per_w = n_total // nw
    assert per_w == _CHUNK * _PHASES
    mesh = plsc.VectorSubcoreMesh(core_axis_name="c", subcore_axis_name="s")

    @functools.partial(
        pl.kernel,
        out_type=jax.ShapeDtypeStruct((n_total, dim), jnp.float32),
        mesh=mesh,
        compiler_params=pltpu.CompilerParams(use_tc_tiling_on_sc=False),
        scratch_types=[
            pltpu.VMEM((per_w,), jnp.int32),
            pltpu.VMEM((2, _CHUNK, dim), jnp.float32),
            pltpu.SemaphoreType.DMA,
            pltpu.SemaphoreType.DMA,
            pltpu.SemaphoreType.DMA,
        ],
    )
    def k(x_hbm, table_hbm, out_hbm, idx_v, rows_v, gsem, osem0, osem1):
        osem = (osem0, osem1)
        wid = lax.axis_index("s") * nc + lax.axis_index("c")
        base = wid * per_w
        pltpu.sync_copy(x_hbm.at[pl.ds(base, per_w)], idx_v)

        def mod_phase(p):
            def body(j, carry):
                sl = pl.ds(p * _CHUNK + j * _LANES, _LANES)
                v = idx_v[sl]
                idx_v[sl] = lax.rem(v, lax.full_like(v, _HASH_MOD))
                return carry

            lax.fori_loop(0, _CHUNK // _LANES, body, 0)

        def gather_copy(p, b):
            return pltpu.make_async_copy(
                table_hbm.at[idx_v.at[pl.ds(p * _CHUNK, _CHUNK)]],
                rows_v.at[b],
                gsem,
            )

        def write_copy(p, b):
            return pltpu.make_async_copy(
                rows_v.at[b],
                out_hbm.at[pl.ds(base + p * _CHUNK, _CHUNK)],
                osem[b],
            )

        mod_phase(0)
        gather_copy(0, 0).start()
        for p in range(_PHASES):
            b = p % 2
            if p + 1 < _PHASES:
                mod_phase(p + 1)
                gather_copy(p, b).wait()
                if p >= 1:
                    write_copy(p - 1, 1 - b).wait()
                gather_copy(p + 1, 1 - b).start()
            else:
                gather_copy(p, b).wait()
            write_copy(p, b).start()
        write_copy(_PHASES - 2, _PHASES % 2).wait()
        write_copy(_PHASES - 1, (_PHASES - 1) % 2).wait()

    return k


def kernel(x, table):
    n_total = x.size
    out = _build(n_total, table.shape[1])(x.reshape(n_total), table)
    return out.reshape(*x.shape, table.shape[1])
